# Initial kernel scaffold; baseline (speedup 1.0000x reference)
import jax
import jax.numpy as jnp
from jax.experimental import pallas as pl


def kernel(logits, cand_scores, cand_seqs, completed_scores, completed_seqs, completed_length, decoder_context, decoder_rnn1, decoder_rnn2):
    raise NotImplementedError("write your pallas kernel here")


# speedup vs baseline: 2.7676x; 2.7676x over previous
import functools

import jax
import jax.numpy as jnp
from jax import lax
from jax.experimental import pallas as pl
from jax.experimental.pallas import tpu as pltpu
from jax.experimental.pallas import tpu_sc as plsc

BATCH = 64
BEAM = 16
NUM_SYMS = 100000
EOS = NUM_SYMS - 1
SEQ = 32
MAX_SEQ = 64
ATTN = 512
DEC = 1024

ROWS = BATCH * BEAM
K2 = 32
NEG = float("-inf")


def _merge_body(vals_ref, idx_ref, z_ref, eos_ref, cand_ref, cscore_ref,
                clen_ref, cseq_ref, compseq_ref,
                ts_ref, sym_ref, par_ref, ncs_ref, ccs_ref, ccseq_ref,
                ccl_ref, gidx_ref):
    vals = vals_ref[...]
    vidx = idx_ref[...]
    z = z_ref[...]
    cand = cand_ref[...]

    col = lax.broadcasted_iota(jnp.int32, (BATCH, BEAM * K2), 1)
    row = col // K2
    zr = jnp.zeros((BATCH, BEAM * K2), jnp.float32)
    cr = jnp.zeros((BATCH, BEAM * K2), jnp.float32)
    for p in range(BEAM):
        m = row == p
        zr = zr + jnp.where(m, z[:, p][:, None], 0.0)
        cr = cr + jnp.where(m, cand[:, p][:, None], 0.0)
    s = (vals - zr) + cr
    flat = row * EOS + vidx

    big = jnp.int32(2**31 - 1)
    ts_cols, sym_cols, par_cols = [], [], []
    for _ in range(BEAM):
        cur = jnp.max(s, axis=1, keepdims=True)
        sel = s == cur
        fmin = jnp.min(jnp.where(sel, flat, big), axis=1, keepdims=True)
        hit = flat == fmin
        ts_cols.append(cur)
        sym_cols.append(jnp.sum(jnp.where(hit, vidx, 0), axis=1, keepdims=True))
        par_cols.append(jnp.sum(jnp.where(hit, row, 0), axis=1, keepdims=True))
        s = jnp.where(hit, NEG, s)
    top_scores = jnp.concatenate(ts_cols, axis=1)
    syms = jnp.concatenate(sym_cols, axis=1)
    parents = jnp.concatenate(par_cols, axis=1)

    ts_ref[...] = top_scores
    sym_ref[...] = syms
    par_ref[...] = parents
    b_iota = lax.broadcasted_iota(jnp.int32, (BATCH, BEAM), 0)
    gidx_ref[...] = b_iota * BEAM + parents

    for k in range(BEAM):
        pk = parents[:, k:k + 1]
        out_k = jnp.zeros((BATCH, SEQ), jnp.int32)
        for p in range(BEAM):
            out_k = out_k + jnp.where(pk == p, cseq_ref[p], 0)
        ncs_ref[k] = out_k

    eosv = eos_ref[...]
    new_eos = (eosv - z) + cand
    cs = jnp.concatenate([cscore_ref[...], new_eos], axis=1)
    cl = jnp.concatenate(
        [clen_ref[...], jnp.full((BATCH, BEAM), SEQ + 1, jnp.int32)], axis=1)
    resc = cs / cl.astype(jnp.float32)
    ccol = lax.broadcasted_iota(jnp.int32, (BATCH, 2 * BEAM), 1)
    ccs_cols, ccl_cols, cidx_cols = [], [], []
    r = resc
    for _ in range(BEAM):
        cur = jnp.max(r, axis=1, keepdims=True)
        sel = r == cur
        imin = jnp.min(jnp.where(sel, ccol, big), axis=1, keepdims=True)
        hit = ccol == imin
        ccs_cols.append(jnp.sum(jnp.where(hit, cs, 0.0), axis=1, keepdims=True))
        ccl_cols.append(jnp.sum(jnp.where(hit, cl, 0), axis=1, keepdims=True))
        cidx_cols.append(imin)
        r = jnp.where(hit, NEG, r)
    ccs_ref[...] = jnp.concatenate(ccs_cols, axis=1)
    ccl_ref[...] = jnp.concatenate(ccl_cols, axis=1)
    cinds = jnp.concatenate(cidx_cols, axis=1)

    eos_pad = jnp.full((BATCH, MAX_SEQ - SEQ), EOS, jnp.int32)
    for k in range(BEAM):
        ck = cinds[:, k:k + 1]
        out_k = jnp.zeros((BATCH, MAX_SEQ), jnp.int32)
        for i in range(2 * BEAM):
            if i < BEAM:
                src = compseq_ref[i]
            else:
                src = jnp.concatenate([cseq_ref[i - BEAM], eos_pad], axis=1)
            out_k = out_k + jnp.where(ck == i, src, 0)
        ccseq_ref[k] = out_k


def _merge(vals, vidx, z, eosv, cand_scores, completed_scores,
           completed_length, cand_seqs, completed_seqs):
    outs = (
        jax.ShapeDtypeStruct((BATCH, BEAM), jnp.float32),
        jax.ShapeDtypeStruct((BATCH, BEAM), jnp.int32),
        jax.ShapeDtypeStruct((BATCH, BEAM), jnp.int32),
        jax.ShapeDtypeStruct((BEAM, BATCH, SEQ), jnp.int32),
        jax.ShapeDtypeStruct((BATCH, BEAM), jnp.float32),
        jax.ShapeDtypeStruct((BEAM, BATCH, MAX_SEQ), jnp.int32),
        jax.ShapeDtypeStruct((BATCH, BEAM), jnp.int32),
        jax.ShapeDtypeStruct((BATCH, BEAM), jnp.int32),
    )
    return pl.pallas_call(_merge_body, out_shape=outs)(
        vals, vidx, z, eosv, cand_scores, completed_scores,
        completed_length, cand_seqs, completed_seqs)


_GROUP = 25
_NGROUPS = 250
_NEGV = float("-inf")


def _lane_gather(x, idx):
    return lax.gather(
        x, idx[:, None],
        lax.GatherDimensionNumbers(offset_dims=(), collapsed_slice_dims=(0,),
                                   start_index_map=(0,)),
        (1,), mode=lax.GatherScatterMode.PROMISE_IN_BOUNDS)


def _bcast0(x):
    return _lane_gather(x, jnp.zeros((16,), jnp.int32))


def _sc_merge_vreg(v, base_idx, st):
    t0v, t0i, t1v, t1i, thr = st
    lane = lax.iota(jnp.int32, 16)
    vd, vdi = plsc.sort_key_val(v, base_idx + lane, descending=True)
    ge = t0v >= vd
    hv = jnp.where(ge, t0v, vd)
    hi = jnp.where(ge, t0i, vdi)
    hs, his = plsc.sort_key_val(hv, hi)
    t1r = lax.rev(t1v, (0,))
    t1ir = lax.rev(t1i, (0,))
    ge2 = hs >= t1r
    lov = jnp.where(ge2, t1r, hs)
    loi = jnp.where(ge2, t1ir, his)
    hiv = jnp.where(ge2, hs, t1r)
    hii = jnp.where(ge2, his, t1ir)
    nt0v, nt0i = plsc.sort_key_val(lov, loi)
    nt1v, nt1i = plsc.sort_key_val(hiv, hii)
    return nt0v, nt0i, nt1v, nt1i, _bcast0(nt0v)


def _row_topk(la):
    info = plsc.get_sparse_core_info()
    nc, ns = info.num_cores, info.num_subcores
    nw = nc * ns
    rpw = ROWS // nw
    mesh = plsc.VectorSubcoreMesh(core_axis_name="c", subcore_axis_name="s")

    @functools.partial(
        pl.kernel, mesh=mesh,
        compiler_params=pltpu.CompilerParams(needs_layout_passes=False),
        out_type=(
            jax.ShapeDtypeStruct((ROWS, K2), jnp.float32),
            jax.ShapeDtypeStruct((ROWS, K2), jnp.int32),
            jax.ShapeDtypeStruct((ROWS,), jnp.float32),
        ),
        scratch_types=[
            pltpu.VMEM((NUM_SYMS,), jnp.float32),
            pltpu.VMEM((rpw, K2), jnp.float32),
            pltpu.VMEM((rpw, K2), jnp.int32),
            pltpu.VMEM((rpw,), jnp.float32),
        ],
    )
    def k(la_hbm, ovals, oidx, oeos, rowbuf, vbuf, ibuf, ebuf):
        wid = lax.axis_index("s") * nc + lax.axis_index("c")
        base = wid * rpw
        lane = lax.iota(jnp.int32, 16)

        def row_body(rl, eacc):
            eacc0, eacc1 = eacc
            pltpu.sync_copy(la_hbm.at[base + rl], rowbuf)
            vlast = rowbuf[pl.ds(NUM_SYMS - 16, 16)]
            eos_splat = _lane_gather(vlast, jnp.full((16,), 15, jnp.int32))
            rowbuf[pl.ds(NUM_SYMS - 16, 16)] = jnp.where(lane < 15, vlast, _NEGV)

            ninf = jnp.full((16,), _NEGV, jnp.float32)
            zi = jnp.zeros((16,), jnp.int32)

            def group_body(g, st):
                gbase = g * (_GROUP * 16)
                vs = [rowbuf[pl.ds(gbase + u * 16, 16)] for u in range(_GROUP)]
                gm = vs[0]
                for u in range(1, _GROUP):
                    gm = jnp.maximum(gm, vs[u])

                def merge_path(st):
                    for u in range(_GROUP):
                        st = lax.cond(jnp.any(vs[u] > st[4]),
                                      lambda s, uu=u: _sc_merge_vreg(
                                          vs[uu], gbase + uu * 16, s),
                                      lambda s: s, st)
                    return st

                return lax.cond(jnp.any(gm > st[4]), merge_path,
                                lambda s: s, st)

            t0v, t0i, t1v, t1i, thr = lax.fori_loop(
                0, _NGROUPS, group_body, (ninf, zi, ninf, zi, ninf))

            vbuf[rl, pl.ds(0, 16)] = t0v
            vbuf[rl, pl.ds(16, 16)] = t1v
            ibuf[rl, pl.ds(0, 16)] = t0i
            ibuf[rl, pl.ds(16, 16)] = t1i
            rlm = rl - (rl // 16) * 16
            eacc0 = jnp.where((lane == rlm) & (rl < 16), eos_splat, eacc0)
            eacc1 = jnp.where((lane == rlm) & (rl >= 16), eos_splat, eacc1)
            return (eacc0, eacc1)

        zf = jnp.zeros((16,), jnp.float32)
        eacc0, eacc1 = lax.fori_loop(0, rpw, row_body, (zf, zf))
        ebuf[pl.ds(0, 16)] = eacc0
        ebuf[pl.ds(16, 16)] = eacc1
        pltpu.sync_copy(vbuf, ovals.at[pl.ds(base, rpw)])
        pltpu.sync_copy(ibuf, oidx.at[pl.ds(base, rpw)])
        pltpu.sync_copy(ebuf, oeos.at[pl.ds(base, rpw)])

    return k(la)


def _regather(gidx_flat, ctx, r1, r2):
    info = plsc.get_sparse_core_info()
    nc, ns = info.num_cores, info.num_subcores
    nw = nc * ns
    bpw = ROWS // nw
    mesh = plsc.VectorSubcoreMesh(core_axis_name="c", subcore_axis_name="s")

    @functools.partial(
        pl.kernel, mesh=mesh,
        out_type=(
            jax.ShapeDtypeStruct((ROWS, ATTN), jnp.float32),
            jax.ShapeDtypeStruct((ROWS, DEC), jnp.float32),
            jax.ShapeDtypeStruct((ROWS, DEC), jnp.float32),
        ),
        scratch_types=[
            pltpu.VMEM((bpw,), jnp.int32),
            pltpu.VMEM((bpw, ATTN), jnp.float32),
            pltpu.VMEM((bpw, DEC), jnp.float32),
            pltpu.SemaphoreType.DMA,
        ],
    )
    def k(idx_hbm, ctx_hbm, r1_hbm, r2_hbm, octx, or1, or2,
          idx_v, bufa, bufd, sem):
        wid = lax.axis_index("s") * nc + lax.axis_index("c")
        base = wid * bpw
        pltpu.sync_copy(idx_hbm.at[pl.ds(base, bpw)], idx_v)
        pltpu.async_copy(ctx_hbm.at[idx_v], bufa, sem).wait()
        pltpu.sync_copy(bufa, octx.at[pl.ds(base, bpw)])
        pltpu.async_copy(r1_hbm.at[idx_v], bufd, sem).wait()
        pltpu.sync_copy(bufd, or1.at[pl.ds(base, bpw)])
        pltpu.async_copy(r2_hbm.at[idx_v], bufd, sem).wait()
        pltpu.sync_copy(bufd, or2.at[pl.ds(base, bpw)])

    return k(gidx_flat, ctx, r1, r2)


def kernel(logits, cand_scores, cand_seqs, completed_scores, completed_seqs,
           completed_length, decoder_context, decoder_rnn1, decoder_rnn2):
    la = logits.reshape(ROWS, NUM_SYMS)

    Z = jax.scipy.special.logsumexp(logits, axis=2)

    vals, vidx, eosf = _row_topk(la)
    vals = vals.reshape(BATCH, BEAM * K2)
    vidx = vidx.reshape(BATCH, BEAM * K2)
    eosv = eosf.reshape(BATCH, BEAM)

    (top_scores, syms, parents, ncs_t, comp_scores, comp_seqs_t, comp_len,
     gidx) = _merge(vals, vidx, Z, eosv, cand_scores, completed_scores,
                    completed_length, cand_seqs.transpose(1, 0, 2),
                    completed_seqs.transpose(1, 0, 2))

    ncs = ncs_t.transpose(1, 0, 2)
    comp_seqs = comp_seqs_t.transpose(1, 0, 2)
    new_cand_seqs = jnp.concatenate([ncs, syms[:, :, None]], axis=2)

    ctx, r1, r2 = _regather(gidx.reshape(ROWS), decoder_context,
                            decoder_rnn1, decoder_rnn2)

    return (top_scores, syms, parents, new_cand_seqs, comp_scores,
            comp_seqs, comp_len, ctx, r1, r2)
